# baseline (speedup 1.0000x reference)
import functools

import jax
import jax.numpy as jnp
from jax import lax
from jax.experimental import pallas as pl
from jax.experimental.pallas import tpu as pltpu
from jax.experimental.pallas import tpu_sc as plsc

N = 10000
E = 100000
D = 768
NP = 10240
R = 1024
NBLK = NP // R
CW = 128
NCH = D // CW
NC, NS = 2, 16
CPC = NCH // NC
EB = 128
ET = 6400
EP = ET * NS
NBE = ET // EB
DUMMY = 10016
ETD = EP // (NC * NS)
NBD = ETD // EB
CB = 2560
CT = CB // NS
ZR = 64
SLICE = NP // NS

@functools.lru_cache(maxsize=None)
def _get_mesh():
    return plsc.VectorSubcoreMesh(core_axis_name="c", subcore_axis_name="s",
                                  num_cores=NC, num_subcores=NS)


def _zero_vec(ref, n):
    z = jnp.zeros((16,), jnp.float32)
    for i in range(n // 16):
        ref[pl.ds(16 * i, 16)] = z


def _deg_body(dst_hbm, out_hbm, idx_b, ones_b, bounce, deg_sh, sem):
    c = lax.axis_index("c")
    s = lax.axis_index("s")
    w = c * NS + s
    for i in range(EB // 16):
        ones_b[pl.ds(16 * i, 16)] = jnp.ones((16,), jnp.float32)
    _zero_vec(bounce, SLICE)
    pltpu.sync_copy(bounce, deg_sh.at[pl.ds(s * SLICE, SLICE)])
    pltpu.sync_copy(dst_hbm.at[w], idx_b)
    plsc.subcore_barrier()

    def batch(b, carry):
        pltpu.sync_copy(ones_b, deg_sh.at[idx_b.at[b]], add=True)
        return carry

    lax.fori_loop(0, NBD, batch, 0)
    plsc.subcore_barrier()
    pltpu.sync_copy(deg_sh.at[pl.ds(s * SLICE, SLICE)], bounce)
    pltpu.sync_copy(bounce, out_hbm.at[c, pl.ds(s * SLICE, SLICE)])


@functools.lru_cache(maxsize=None)
def _deg_call():
    return pl.kernel(
        _deg_body,
        out_type=jax.ShapeDtypeStruct((NC, NP), jnp.float32),
        mesh=_get_mesh(),
        scratch_types=[
            pltpu.VMEM((NBD, EB), jnp.int32),
            pltpu.VMEM((EB,), jnp.float32),
            pltpu.VMEM((SLICE,), jnp.float32),
            pltpu.VMEM_SHARED((NP,), jnp.float32),
            pltpu.SemaphoreType.DMA,
        ],
    )


NRING = 2
NGRP = NBE // NRING


def _edge_chunk_loop(y_hbm, gsrc_hbm, dst3_hbm, gidx2, didx2, gb,
                     acc_sh, gsem, ssem, epilogue):
    c = lax.axis_index("c")
    s = lax.axis_index("s")

    pltpu.sync_copy(dst3_hbm.at[s], didx2)

    def issue_g(b, i):
        pltpu.async_copy(y_hbm.at[gidx2.at[b]], gb[i], gsem.at[i])

    def wait_g(i):
        pltpu.make_async_copy(y_hbm.at[gidx2.at[0]], gb[i],
                              gsem.at[i]).wait()

    def issue_s(b, i):
        pltpu.async_copy(gb[i], acc_sh.at[didx2.at[b]], ssem.at[i],
                         add=True)

    def wait_s(i):
        pltpu.make_async_copy(gb[i], acc_sh.at[didx2.at[0]],
                              ssem.at[i]).wait()

    for t in range(CPC):
        j = c * CPC + t

        def zrow(r, carry):
            z = jnp.zeros((16,), jnp.float32)
            for i in range(CW // 16):
                gb[0][r, pl.ds(16 * i, 16)] = z
            return carry

        lax.fori_loop(0, EB, zrow, 0)

        def zslab(z, carry):
            pltpu.sync_copy(gb[0], acc_sh.at[pl.ds(s * SLICE + z * EB, EB)])
            return carry

        lax.fori_loop(0, SLICE // EB, zslab, 0)
        pltpu.sync_copy(gsrc_hbm.at[j, s], gidx2)
        plsc.subcore_barrier()

        issue_g(0, 0)

        def group(g, carry):
            b0 = g * NRING
            for k in range(NRING):
                b = b0 + k
                wait_g(k)
                issue_s(b, k)

                @pl.when(b >= 1)
                def _ws():
                    wait_s((k + 1) % NRING)

                @pl.when(b + 1 < NBE)
                def _ig():
                    issue_g(b + 1, (k + 1) % NRING)
            return carry

        lax.fori_loop(0, NGRP, group, 0)
        wait_s((NBE - 1) % NRING)
        plsc.subcore_barrier()
        epilogue(j, c, s)
        plsc.subcore_barrier()


def _pass1_body(y_hbm, gsrc_hbm, dst3_hbm, acc_hbm,
                gidx2, didx2, gb0, gb1, acc_sh, gsem, ssem):
    def epilogue(j, c, s):
        r0 = s * SLICE
        pltpu.sync_copy(acc_sh.at[pl.ds(r0, SLICE)],
                        acc_hbm.at[pl.ds(j * NP + r0, SLICE)])

    _edge_chunk_loop(y_hbm, gsrc_hbm, dst3_hbm, gidx2, didx2,
                     [gb0, gb1], acc_sh, gsem, ssem, epilogue)


@functools.lru_cache(maxsize=None)
def _pass1_call():
    return pl.kernel(
        _pass1_body,
        out_type=jax.ShapeDtypeStruct((NCH * NP, CW), jnp.float32),
        mesh=_get_mesh(),
        scratch_types=[
            pltpu.VMEM((NBE, EB), jnp.int32),
            pltpu.VMEM((NBE, EB), jnp.int32),
            pltpu.VMEM((EB, CW), jnp.float32),
            pltpu.VMEM((EB, CW), jnp.float32),
            pltpu.VMEM_SHARED((NP, CW), jnp.float32),
            pltpu.SemaphoreType.DMA((NRING,)),
            pltpu.SemaphoreType.DMA((NRING,)),
        ],
    )


def _pass2_body(y_hbm, gsrc_hbm, dst3_hbm, cidx_hbm, gcidx_hbm, dinv_hbm,
                hacc_hbm, hy_hbm, dinvr_hbm,
                gidx2, didx2, gb0, gb1, cv, cg, dvbuf,
                acc_sh, gsem, ssem):
    c = lax.axis_index("c")
    s = lax.axis_index("s")

    @pl.when(c == 0)
    def _dinv_rows():
        for q in range(2):
            base = s * CT + q * (CT // 2)
            pltpu.sync_copy(cidx_hbm.at[2 * s + q], cv)
            pltpu.async_copy(dinv_hbm.at[cv], dvbuf, gsem.at[0]).wait()
            pltpu.sync_copy(dvbuf, dinvr_hbm.at[pl.ds(base, CT // 2)])

    def epilogue(j, c, s):
        hbuf = gb0.at[pl.ds(0, CT // 2)]
        for q in range(2):
            base = s * CT + q * (CT // 2)
            pltpu.sync_copy(cidx_hbm.at[2 * s + q], cv)
            pltpu.async_copy(acc_sh.at[cv], hbuf, gsem.at[0]).wait()
            pltpu.sync_copy(hbuf, hacc_hbm.at[pl.ds(j * CB + base, CT // 2)])
            pltpu.sync_copy(gcidx_hbm.at[j, 2 * s + q], cg)
            pltpu.async_copy(y_hbm.at[cg], hbuf, gsem.at[0]).wait()
            pltpu.sync_copy(hbuf, hy_hbm.at[pl.ds(j * CB + base, CT // 2)])

    _edge_chunk_loop(y_hbm, gsrc_hbm, dst3_hbm, gidx2, didx2,
                     [gb0, gb1], acc_sh, gsem, ssem, epilogue)


@functools.lru_cache(maxsize=None)
def _pass2_call():
    return pl.kernel(
        _pass2_body,
        out_type=[
            jax.ShapeDtypeStruct((NCH * CB, CW), jnp.float32),
            jax.ShapeDtypeStruct((NCH * CB, CW), jnp.float32),
            jax.ShapeDtypeStruct((CB,), jnp.float32),
        ],
        mesh=_get_mesh(),
        scratch_types=[
            pltpu.VMEM((NBE, EB), jnp.int32),
            pltpu.VMEM((NBE, EB), jnp.int32),
            pltpu.VMEM((EB, CW), jnp.float32),
            pltpu.VMEM((EB, CW), jnp.float32),
            pltpu.VMEM((CT // 2,), jnp.int32),
            pltpu.VMEM((CT // 2,), jnp.int32),
            pltpu.VMEM((CT // 2,), jnp.float32),
            pltpu.VMEM_SHARED((NP, CW), jnp.float32),
            pltpu.SemaphoreType.DMA((NRING,)),
            pltpu.SemaphoreType.DMA((NRING,)),
        ],
    )


def _mm1_body(x_ref, w_ref, deg_ref, y_ref, dinv_ref):
    dinv = lax.rsqrt(deg_ref[0] + deg_ref[1] + 1.0)
    y = jnp.dot(x_ref[...], w_ref[...],
                preferred_element_type=jnp.float32) * dinv[:, None]
    for j in range(NCH):
        y_ref[j] = y[:, j * CW:(j + 1) * CW]
    dinv_ref[...] = dinv


def _mm1(x_pad, W1, indeg2):
    return pl.pallas_call(
        _mm1_body,
        grid=(NBLK,),
        in_specs=[
            pl.BlockSpec((R, D), lambda i: (i, 0)),
            pl.BlockSpec((D, D), lambda i: (0, 0)),
            pl.BlockSpec((NC, R), lambda i: (0, i)),
        ],
        out_specs=[
            pl.BlockSpec((NCH, R, CW), lambda i: (0, i, 0)),
            pl.BlockSpec((R,), lambda i: (i,)),
        ],
        out_shape=[
            jax.ShapeDtypeStruct((NCH, NP, CW), jnp.float32),
            jax.ShapeDtypeStruct((NP,), jnp.float32),
        ],
    )(x_pad, W1, indeg2)


def _mm2_body(acc_ref, y1_ref, dinv_ref, b1_ref, w_ref, y2_ref):
    dinv = dinv_ref[...]
    pre = jnp.concatenate([acc_ref[j] + y1_ref[j] for j in range(NCH)],
                          axis=1)
    h1 = jnp.maximum(pre * dinv[:, None] + b1_ref[...][None, :], 0.0)
    y2 = jnp.dot(h1, w_ref[...],
                 preferred_element_type=jnp.float32) * dinv[:, None]
    for j in range(NCH):
        y2_ref[j] = y2[:, j * CW:(j + 1) * CW]


def _mm2(acc1, y1, dinv, b1, W2):
    return pl.pallas_call(
        _mm2_body,
        grid=(NBLK,),
        in_specs=[
            pl.BlockSpec((NCH, R, CW), lambda i: (0, i, 0)),
            pl.BlockSpec((NCH, R, CW), lambda i: (0, i, 0)),
            pl.BlockSpec((R,), lambda i: (i,)),
            pl.BlockSpec((D,), lambda i: (0,)),
            pl.BlockSpec((D, D), lambda i: (0, 0)),
        ],
        out_specs=pl.BlockSpec((NCH, R, CW), lambda i: (0, i, 0)),
        out_shape=jax.ShapeDtypeStruct((NCH, NP, CW), jnp.float32),
    )(acc1, y1, dinv, b1, W2)


def _score_body(hacc_ref, hy_ref, dinvr_ref, b2_ref, wp_ref, bp_ref, cp_ref,
                out2_ref, out3_ref):
    h = jnp.concatenate([hacc_ref[j] + hy_ref[j] for j in range(NCH)],
                        axis=1)
    h = h * dinvr_ref[...][:, None] + b2_ref[...][None, :]
    G = jnp.dot(h, wp_ref[...], preferred_element_type=jnp.float32)
    bp = bp_ref[...][None, :]
    hb = jnp.sum(h * bp, axis=1, keepdims=True)
    rowdot = jnp.sum(G * h, axis=1, keepdims=True)
    cp = cp_ref[0]

    def segsum(m, lo, cnt):
        rows = jax.lax.broadcasted_iota(jnp.int32, (512, 512 * cnt), 0)
        cols = jax.lax.broadcasted_iota(jnp.int32, (512, 512 * cnt), 1)
        S = (cols // cnt == rows).astype(jnp.float32)
        return jnp.dot(S, m[lo:lo + 512 * cnt],
                       preferred_element_type=jnp.float32)

    def score(lo, cnt):
        hs = segsum(h, lo, cnt)
        Gs = segsum(G, lo, cnt)
        tot = jnp.sum(Gs * hs, axis=1)
        diag = segsum(rowdot, lo, cnt)[:, 0]
        first = segsum(hb, lo, cnt)[:, 0] / cnt
        second = (tot - diag) / (cnt * (cnt - 1))
        z = first + second + cp
        return 1.0 / (1.0 + jnp.exp(-z))

    out2_ref[...] = score(0, 2)
    out3_ref[...] = score(1024, 3)


def _score(hacc, hy, dinvr, b2, W_p, bp_flat, c_p):
    return pl.pallas_call(
        _score_body,
        in_specs=[
            pl.BlockSpec((NCH, CB, CW), lambda: (0, 0, 0)),
            pl.BlockSpec((NCH, CB, CW), lambda: (0, 0, 0)),
            pl.BlockSpec((CB,), lambda: (0,)),
            pl.BlockSpec((D,), lambda: (0,)),
            pl.BlockSpec((D, D), lambda: (0, 0)),
            pl.BlockSpec((D,), lambda: (0,)),
            pl.BlockSpec(memory_space=pltpu.SMEM),
        ],
        out_specs=[
            pl.BlockSpec((512,), lambda: (0,)),
            pl.BlockSpec((512,), lambda: (0,)),
        ],
        out_shape=[
            jax.ShapeDtypeStruct((512,), jnp.float32),
            jax.ShapeDtypeStruct((512,), jnp.float32),
        ],
    )(hacc, hy, dinvr, b2, W_p, bp_flat, c_p)


def kernel(x, edge_index, comb2, comb3, W1, b1, W2, b2, b_p, W_p, c_p):
    src = jnp.concatenate([edge_index[0],
                           jnp.zeros((EP - E,), jnp.int32)])
    dst = jnp.concatenate([edge_index[1],
                           jnp.full((EP - E,), DUMMY, jnp.int32)])
    x_pad = jnp.pad(x, ((0, NP - N), (0, 0)))
    cidx = jnp.concatenate([comb2.reshape(-1), comb3.reshape(-1)])

    coff = (jnp.arange(NCH, dtype=jnp.int32) * NP)
    src3 = src.reshape(NS, NBE, EB)
    dst3 = dst.reshape(NS, NBE, EB)
    gsrc = src3[None] + coff[:, None, None, None]
    dst4 = dst.reshape(NC * NS, NBD, EB)
    cidx2 = cidx.reshape(2 * NS, CT // 2)
    gcidx = cidx2[None] + coff[:, None, None]

    indeg2 = _deg_call()(dst4)
    y1_3d, dinv = _mm1(x_pad, W1, indeg2)
    acc1 = _pass1_call()(y1_3d.reshape(NCH * NP, CW), gsrc, dst3)
    y2_3d = _mm2(acc1.reshape(NCH, NP, CW), y1_3d, dinv, b1, W2)
    hacc, hy, dinvr = _pass2_call()(y2_3d.reshape(NCH * NP, CW), gsrc, dst3,
                                    cidx2, gcidx, dinv)
    out2, out3 = _score(hacc.reshape(NCH, CB, CW), hy.reshape(NCH, CB, CW),
                        dinvr, b2, W_p, b_p.reshape(-1), c_p)
    return jnp.concatenate([out2, out3])


# speedup vs baseline: 1.0016x; 1.0016x over previous
import functools

import jax
import jax.numpy as jnp
from jax import lax
from jax.experimental import pallas as pl
from jax.experimental.pallas import tpu as pltpu
from jax.experimental.pallas import tpu_sc as plsc

N = 10000
E = 100000
D = 768
NP = 10240
R = 1024
NBLK = NP // R
CW = 128
NCH = D // CW
NC, NS = 2, 16
CPC = NCH // NC
EB = 128
ET = 6400
EP = ET * NS
NBE = ET // EB
DUMMY = 10016
ETD = EP // (NC * NS)
NBD = ETD // EB
CB = 2560
CT = CB // NS
ZR = 64
SLICE = NP // NS

@functools.lru_cache(maxsize=None)
def _get_mesh():
    return plsc.VectorSubcoreMesh(core_axis_name="c", subcore_axis_name="s",
                                  num_cores=NC, num_subcores=NS)


def _zero_vec(ref, n):
    z = jnp.zeros((16,), jnp.float32)
    for i in range(n // 16):
        ref[pl.ds(16 * i, 16)] = z


def _deg_body(dst_hbm, out_hbm, idx_b, ones_b, bounce, deg_sh, sem):
    c = lax.axis_index("c")
    s = lax.axis_index("s")
    w = c * NS + s
    for i in range(EB // 16):
        ones_b[pl.ds(16 * i, 16)] = jnp.ones((16,), jnp.float32)
    _zero_vec(bounce, SLICE)
    pltpu.sync_copy(bounce, deg_sh.at[pl.ds(s * SLICE, SLICE)])
    pltpu.sync_copy(dst_hbm.at[w], idx_b)
    plsc.subcore_barrier()

    def batch(b, carry):
        pltpu.sync_copy(ones_b, deg_sh.at[idx_b.at[b]], add=True)
        return carry

    lax.fori_loop(0, NBD, batch, 0)
    plsc.subcore_barrier()
    pltpu.sync_copy(deg_sh.at[pl.ds(s * SLICE, SLICE)], bounce)
    pltpu.sync_copy(bounce, out_hbm.at[c, pl.ds(s * SLICE, SLICE)])


@functools.lru_cache(maxsize=None)
def _deg_call():
    return pl.kernel(
        _deg_body,
        out_type=jax.ShapeDtypeStruct((NC, NP), jnp.float32),
        mesh=_get_mesh(),
        scratch_types=[
            pltpu.VMEM((NBD, EB), jnp.int32),
            pltpu.VMEM((EB,), jnp.float32),
            pltpu.VMEM((SLICE,), jnp.float32),
            pltpu.VMEM_SHARED((NP,), jnp.float32),
            pltpu.SemaphoreType.DMA,
        ],
    )


NRING = 2


def _edge_chunk_loop(y_hbm, gsrc_hbm, dst3_hbm, gidx2, didx2, gb,
                     acc_sh, gsem, epilogue):
    c = lax.axis_index("c")
    s = lax.axis_index("s")

    pltpu.sync_copy(dst3_hbm.at[s], didx2)

    def issue(b, i):
        pltpu.async_copy(y_hbm.at[gidx2.at[b]], gb[i], gsem.at[i])

    def wait_gather(i):
        pltpu.make_async_copy(y_hbm.at[gidx2.at[0]], gb[i],
                              gsem.at[i]).wait()

    def do_scatter(b, i):
        pltpu.sync_copy(gb[i], acc_sh.at[didx2.at[b]], add=True)

    for t in range(CPC):
        j = c * CPC + t

        def zrow(r, carry):
            z = jnp.zeros((16,), jnp.float32)
            for i in range(CW // 16):
                gb[0][r, pl.ds(16 * i, 16)] = z
            return carry

        lax.fori_loop(0, EB, zrow, 0)

        def zslab(z, carry):
            pltpu.sync_copy(gb[0], acc_sh.at[pl.ds(s * SLICE + z * EB, EB)])
            return carry

        lax.fori_loop(0, SLICE // EB, zslab, 0)
        pltpu.sync_copy(gsrc_hbm.at[j, s], gidx2)
        plsc.subcore_barrier()

        issue(0, 0)

        def super_it(g, carry):
            b0 = g * NRING
            for i in range(NRING):
                wait_gather(i)
                issue(b0 + i + 1, (i + 1) % NRING)
                do_scatter(b0 + i, i)
            return carry

        lax.fori_loop(0, (NBE - NRING) // NRING, super_it, 0)
        for b in range(NBE - NRING, NBE):
            wait_gather(b % NRING)
            if b + 1 < NBE:
                issue(b + 1, (b + 1) % NRING)
            do_scatter(b, b % NRING)
        plsc.subcore_barrier()
        epilogue(j, c, s)
        plsc.subcore_barrier()


def _pass1_body(y_hbm, gsrc_hbm, dst3_hbm, acc_hbm,
                gidx2, didx2, gb0, gb1, acc_sh, gsem):
    def epilogue(j, c, s):
        r0 = s * SLICE
        pltpu.sync_copy(acc_sh.at[pl.ds(r0, SLICE)],
                        acc_hbm.at[pl.ds(j * NP + r0, SLICE)])

    _edge_chunk_loop(y_hbm, gsrc_hbm, dst3_hbm, gidx2, didx2,
                     [gb0, gb1], acc_sh, gsem, epilogue)


@functools.lru_cache(maxsize=None)
def _pass1_call():
    return pl.kernel(
        _pass1_body,
        out_type=jax.ShapeDtypeStruct((NCH * NP, CW), jnp.float32),
        mesh=_get_mesh(),
        scratch_types=[
            pltpu.VMEM((NBE, EB), jnp.int32),
            pltpu.VMEM((NBE, EB), jnp.int32),
            pltpu.VMEM((EB, CW), jnp.float32),
            pltpu.VMEM((EB, CW), jnp.float32),
            pltpu.VMEM_SHARED((NP, CW), jnp.float32),
            pltpu.SemaphoreType.DMA((NRING,)),
        ],
    )


def _pass2_body(y_hbm, gsrc_hbm, dst3_hbm, cidx_hbm, gcidx_hbm, dinv_hbm,
                hacc_hbm, hy_hbm, dinvr_hbm,
                gidx2, didx2, gb0, gb1, cv, cg, dvbuf,
                acc_sh, gsem):
    c = lax.axis_index("c")
    s = lax.axis_index("s")

    @pl.when(c == 0)
    def _dinv_rows():
        for q in range(2):
            base = s * CT + q * (CT // 2)
            pltpu.sync_copy(cidx_hbm.at[2 * s + q], cv)
            pltpu.async_copy(dinv_hbm.at[cv], dvbuf, gsem.at[0]).wait()
            pltpu.sync_copy(dvbuf, dinvr_hbm.at[pl.ds(base, CT // 2)])

    def epilogue(j, c, s):
        hbuf = gb0.at[pl.ds(0, CT // 2)]
        for q in range(2):
            base = s * CT + q * (CT // 2)
            pltpu.sync_copy(cidx_hbm.at[2 * s + q], cv)
            pltpu.async_copy(acc_sh.at[cv], hbuf, gsem.at[0]).wait()
            pltpu.sync_copy(hbuf, hacc_hbm.at[pl.ds(j * CB + base, CT // 2)])
            pltpu.sync_copy(gcidx_hbm.at[j, 2 * s + q], cg)
            pltpu.async_copy(y_hbm.at[cg], hbuf, gsem.at[0]).wait()
            pltpu.sync_copy(hbuf, hy_hbm.at[pl.ds(j * CB + base, CT // 2)])

    _edge_chunk_loop(y_hbm, gsrc_hbm, dst3_hbm, gidx2, didx2,
                     [gb0, gb1], acc_sh, gsem, epilogue)


@functools.lru_cache(maxsize=None)
def _pass2_call():
    return pl.kernel(
        _pass2_body,
        out_type=[
            jax.ShapeDtypeStruct((NCH * CB, CW), jnp.float32),
            jax.ShapeDtypeStruct((NCH * CB, CW), jnp.float32),
            jax.ShapeDtypeStruct((CB,), jnp.float32),
        ],
        mesh=_get_mesh(),
        scratch_types=[
            pltpu.VMEM((NBE, EB), jnp.int32),
            pltpu.VMEM((NBE, EB), jnp.int32),
            pltpu.VMEM((EB, CW), jnp.float32),
            pltpu.VMEM((EB, CW), jnp.float32),
            pltpu.VMEM((CT // 2,), jnp.int32),
            pltpu.VMEM((CT // 2,), jnp.int32),
            pltpu.VMEM((CT // 2,), jnp.float32),
            pltpu.VMEM_SHARED((NP, CW), jnp.float32),
            pltpu.SemaphoreType.DMA((NRING,)),
        ],
    )


def _mm1_body(x_ref, w_ref, deg_ref, y_ref, dinv_ref):
    dinv = lax.rsqrt(deg_ref[0] + deg_ref[1] + 1.0)
    y = jnp.dot(x_ref[...], w_ref[...],
                preferred_element_type=jnp.float32) * dinv[:, None]
    for j in range(NCH):
        y_ref[j] = y[:, j * CW:(j + 1) * CW]
    dinv_ref[...] = dinv


def _mm1(x_pad, W1, indeg2):
    return pl.pallas_call(
        _mm1_body,
        grid=(NBLK,),
        in_specs=[
            pl.BlockSpec((R, D), lambda i: (i, 0)),
            pl.BlockSpec((D, D), lambda i: (0, 0)),
            pl.BlockSpec((NC, R), lambda i: (0, i)),
        ],
        out_specs=[
            pl.BlockSpec((NCH, R, CW), lambda i: (0, i, 0)),
            pl.BlockSpec((R,), lambda i: (i,)),
        ],
        out_shape=[
            jax.ShapeDtypeStruct((NCH, NP, CW), jnp.float32),
            jax.ShapeDtypeStruct((NP,), jnp.float32),
        ],
    )(x_pad, W1, indeg2)


def _mm2_body(acc_ref, y1_ref, dinv_ref, b1_ref, w_ref, y2_ref):
    dinv = dinv_ref[...]
    pre = jnp.concatenate([acc_ref[j] + y1_ref[j] for j in range(NCH)],
                          axis=1)
    h1 = jnp.maximum(pre * dinv[:, None] + b1_ref[...][None, :], 0.0)
    y2 = jnp.dot(h1, w_ref[...],
                 preferred_element_type=jnp.float32) * dinv[:, None]
    for j in range(NCH):
        y2_ref[j] = y2[:, j * CW:(j + 1) * CW]


def _mm2(acc1, y1, dinv, b1, W2):
    return pl.pallas_call(
        _mm2_body,
        grid=(NBLK,),
        in_specs=[
            pl.BlockSpec((NCH, R, CW), lambda i: (0, i, 0)),
            pl.BlockSpec((NCH, R, CW), lambda i: (0, i, 0)),
            pl.BlockSpec((R,), lambda i: (i,)),
            pl.BlockSpec((D,), lambda i: (0,)),
            pl.BlockSpec((D, D), lambda i: (0, 0)),
        ],
        out_specs=pl.BlockSpec((NCH, R, CW), lambda i: (0, i, 0)),
        out_shape=jax.ShapeDtypeStruct((NCH, NP, CW), jnp.float32),
    )(acc1, y1, dinv, b1, W2)


def _score_body(hacc_ref, hy_ref, dinvr_ref, b2_ref, wp_ref, bp_ref, cp_ref,
                out2_ref, out3_ref):
    h = jnp.concatenate([hacc_ref[j] + hy_ref[j] for j in range(NCH)],
                        axis=1)
    h = h * dinvr_ref[...][:, None] + b2_ref[...][None, :]
    G = jnp.dot(h, wp_ref[...], preferred_element_type=jnp.float32)
    bp = bp_ref[...][None, :]
    hb = jnp.sum(h * bp, axis=1, keepdims=True)
    rowdot = jnp.sum(G * h, axis=1, keepdims=True)
    cp = cp_ref[0]

    def segsum(m, lo, cnt):
        rows = jax.lax.broadcasted_iota(jnp.int32, (512, 512 * cnt), 0)
        cols = jax.lax.broadcasted_iota(jnp.int32, (512, 512 * cnt), 1)
        S = (cols // cnt == rows).astype(jnp.float32)
        return jnp.dot(S, m[lo:lo + 512 * cnt],
                       preferred_element_type=jnp.float32)

    def score(lo, cnt):
        hs = segsum(h, lo, cnt)
        Gs = segsum(G, lo, cnt)
        tot = jnp.sum(Gs * hs, axis=1)
        diag = segsum(rowdot, lo, cnt)[:, 0]
        first = segsum(hb, lo, cnt)[:, 0] / cnt
        second = (tot - diag) / (cnt * (cnt - 1))
        z = first + second + cp
        return 1.0 / (1.0 + jnp.exp(-z))

    out2_ref[...] = score(0, 2)
    out3_ref[...] = score(1024, 3)


def _score(hacc, hy, dinvr, b2, W_p, bp_flat, c_p):
    return pl.pallas_call(
        _score_body,
        in_specs=[
            pl.BlockSpec((NCH, CB, CW), lambda: (0, 0, 0)),
            pl.BlockSpec((NCH, CB, CW), lambda: (0, 0, 0)),
            pl.BlockSpec((CB,), lambda: (0,)),
            pl.BlockSpec((D,), lambda: (0,)),
            pl.BlockSpec((D, D), lambda: (0, 0)),
            pl.BlockSpec((D,), lambda: (0,)),
            pl.BlockSpec(memory_space=pltpu.SMEM),
        ],
        out_specs=[
            pl.BlockSpec((512,), lambda: (0,)),
            pl.BlockSpec((512,), lambda: (0,)),
        ],
        out_shape=[
            jax.ShapeDtypeStruct((512,), jnp.float32),
            jax.ShapeDtypeStruct((512,), jnp.float32),
        ],
    )(hacc, hy, dinvr, b2, W_p, bp_flat, c_p)


def kernel(x, edge_index, comb2, comb3, W1, b1, W2, b2, b_p, W_p, c_p):
    src = jnp.concatenate([edge_index[0],
                           jnp.zeros((EP - E,), jnp.int32)])
    dst = jnp.concatenate([edge_index[1],
                           jnp.full((EP - E,), DUMMY, jnp.int32)])
    x_pad = jnp.pad(x, ((0, NP - N), (0, 0)))
    cidx = jnp.concatenate([comb2.reshape(-1), comb3.reshape(-1)])

    coff = (jnp.arange(NCH, dtype=jnp.int32) * NP)
    src3 = src.reshape(NS, NBE, EB)
    dst3 = dst.reshape(NS, NBE, EB)
    gsrc = src3[None] + coff[:, None, None, None]
    dst4 = dst.reshape(NC * NS, NBD, EB)
    cidx2 = cidx.reshape(2 * NS, CT // 2)
    gcidx = cidx2[None] + coff[:, None, None]

    indeg2 = _deg_call()(dst4)
    y1_3d, dinv = _mm1(x_pad, W1, indeg2)
    acc1 = _pass1_call()(y1_3d.reshape(NCH * NP, CW), gsrc, dst3)
    y2_3d = _mm2(acc1.reshape(NCH, NP, CW), y1_3d, dinv, b1, W2)
    hacc, hy, dinvr = _pass2_call()(y2_3d.reshape(NCH * NP, CW), gsrc, dst3,
                                    cidx2, gcidx, dinv)
    out2, out3 = _score(hacc.reshape(NCH, CB, CW), hy.reshape(NCH, CB, CW),
                        dinvr, b2, W_p, b_p.reshape(-1), c_p)
    return jnp.concatenate([out2, out3])
